# Initial kernel scaffold; baseline (speedup 1.0000x reference)
import jax
import jax.numpy as jnp
from jax.experimental import pallas as pl


def kernel(poss_node, poss_edge, groundTruth, mask, edges):
    raise NotImplementedError("write your pallas kernel here")


# speedup vs baseline: 36.8726x; 36.8726x over previous
import functools

import jax
import jax.numpy as jnp
from jax import lax
from jax.experimental import pallas as pl
from jax.experimental.pallas import tpu as pltpu
from jax.experimental.pallas import tpu_sc as plsc

_N = 100000
_E = 3200000
_C = 7
_ROW = 8
_REC = 16
_SEMI_LAMBDA = 0.001
_EDGE_LAMBDA = 1.0

_NC = 2
_NS = 16
_NW = _NC * _NS
_EPW = _E // _NW
_B = 800
_NCHUNK = _EPW // _B
_SUB = 100
_NSUB = _B // _SUB
_STEPS = _B // 16
_NPT = 3136
_NPAD = _NW * _NPT
_NSTEPS = _NPT // 16

_LN2 = 0.6931471805599453


def _vlog(x):
    bits = lax.bitcast_convert_type(x, jnp.int32)
    e = (bits >> 23) - 127
    m = lax.bitcast_convert_type(
        (bits & jnp.int32(0x007FFFFF)) | jnp.int32(0x3F800000), jnp.float32)
    big = m > jnp.float32(1.41421356)
    m = jnp.where(big, m * jnp.float32(0.5), m)
    e = jnp.where(big, e + 1, e)
    f = m - jnp.float32(1.0)
    z = f * f
    p = jnp.float32(7.0376836292e-2)
    p = p * f + jnp.float32(-1.1514610310e-1)
    p = p * f + jnp.float32(1.1676998740e-1)
    p = p * f + jnp.float32(-1.2420140846e-1)
    p = p * f + jnp.float32(1.4249322787e-1)
    p = p * f + jnp.float32(-1.6668057665e-1)
    p = p * f + jnp.float32(2.0000714765e-1)
    p = p * f + jnp.float32(-2.4999993993e-1)
    p = p * f + jnp.float32(3.3333331174e-1)
    r = f * z * p - jnp.float32(0.5) * z + f
    return r + e.astype(jnp.float32) * jnp.float32(_LN2)


def _body(*refs):
    (pe, rect, edg, pnd, gtt, mskt, out) = refs[:7]
    idx = refs[7:7 + 2 * _NSUB]
    own, rec0, rec1, npn, ngt, nmsk, accout, sem = refs[7 + 2 * _NSUB:]

    wid = lax.axis_index("s") * _NC + lax.axis_index("c")
    la = lax.iota(jnp.int32, 16)
    cols = [jnp.full((16,), c, jnp.int32) for c in range(10)]
    zero = jnp.zeros((16,), jnp.float32)
    one = jnp.full((16,), 1.0, jnp.float32)

    nbase = wid * _NPT
    pltpu.sync_copy(pnd.at[pl.ds(nbase, _NPT)], npn)
    pltpu.sync_copy(gtt.at[pl.ds(wid * _NSTEPS, _NSTEPS)], ngt)
    pltpu.sync_copy(mskt.at[pl.ds(wid * _NSTEPS, _NSTEPS)], nmsk)

    def nstep(t, carry):
        nlog, ncnt = carry
        j = t * 16 + la
        g = ngt[t]
        pn = plsc.load_gather(npn, [j, g])
        mv = nmsk[t]
        return nlog + mv * _vlog(pn), ncnt + mv

    nlog, ncnt = lax.fori_loop(0, _NSTEPS, nstep, (zero, zero))

    ebase = wid * _EPW
    rbase = wid * (_EPW // _SUB)

    def chunk(ci, carry):
        erow = rbase + ci * _NSUB
        for jsub in range(_NSUB):
            pltpu.sync_copy(edg.at[0, erow + jsub], idx[jsub])
            pltpu.sync_copy(edg.at[1, erow + jsub], idx[_NSUB + jsub])
        pltpu.sync_copy(pe.at[pl.ds(ebase + ci * _B, _B)], own)
        descs = []
        for jsub in range(_NSUB):
            descs.append(pltpu.async_copy(
                rect.at[idx[jsub]], rec0.at[pl.ds(jsub * _SUB, _SUB)], sem))
            descs.append(pltpu.async_copy(
                rect.at[idx[_NSUB + jsub]], rec1.at[pl.ds(jsub * _SUB, _SUB)],
                sem))
        for d in descs:
            d.wait()

        def step(t, c2):
            s2, slog, sden, slast = c2
            j = t * 16 + la
            acc = jnp.zeros((16,), jnp.float32)
            for c in range(_ROW):
                a = plsc.load_gather(rec0, [j, cols[c]])
                b = plsc.load_gather(rec1, [j, cols[c]])
                d = a - b
                acc = acc + d * d
            g0f = plsc.load_gather(rec0, [j, cols[8]])
            m0f = plsc.load_gather(rec0, [j, cols[9]])
            g1f = plsc.load_gather(rec1, [j, cols[8]])
            m1f = plsc.load_gather(rec1, [j, cols[9]])
            g0 = g0f.astype(jnp.int32)
            g1 = g1f.astype(jnp.int32)
            plast = plsc.load_gather(own, [j, cols[7]])
            p0 = plsc.load_gather(own, [j, g0])
            p1 = plsc.load_gather(own, [j, g1])
            m0b = m0f > jnp.float32(0.5)
            m1b = m1f > jnp.float32(0.5)
            bothb = (m0f * m1f) > jnp.float32(0.5)
            anyf = jnp.maximum(m0f, m1f)
            sameb = g0 == g1
            arg1 = jnp.where(sameb, p0, plast)
            arg2 = jnp.where(m0b, plast + p0,
                             jnp.where(m1b, plast + p1, one))
            arg = jnp.where(bothb, arg1, arg2)
            return (s2 + acc, slog + _vlog(arg), sden + anyf, slast + plast)

        return lax.fori_loop(0, _STEPS, step, carry)

    s2, slog, sden, slast = lax.fori_loop(
        0, _NCHUNK, chunk, (zero, zero, zero, zero))

    accout[0] = s2
    accout[1] = slog
    accout[2] = sden
    accout[3] = slast
    accout[4] = nlog
    accout[5] = ncnt
    pltpu.sync_copy(accout, out.at[wid])


_mesh = plsc.VectorSubcoreMesh(
    core_axis_name="c", subcore_axis_name="s", num_cores=_NC,
    num_subcores=_NS)

_sc_call = pl.kernel(
    _body,
    out_type=jax.ShapeDtypeStruct((_NW, 6, 16), jnp.float32),
    mesh=_mesh,
    scratch_types=(
        [pltpu.VMEM((_SUB,), jnp.int32) for _ in range(2 * _NSUB)]
        + [
            pltpu.VMEM((_B, _ROW), jnp.float32),
            pltpu.VMEM((_B, _REC), jnp.float32),
            pltpu.VMEM((_B, _REC), jnp.float32),
            pltpu.VMEM((_NPT, _ROW), jnp.float32),
            pltpu.VMEM((_NSTEPS, 16), jnp.int32),
            pltpu.VMEM((_NSTEPS, 16), jnp.float32),
            pltpu.VMEM((6, 16), jnp.float32),
            pltpu.SemaphoreType.DMA,
        ]),
    compiler_params=pltpu.CompilerParams(
        use_tc_tiling_on_sc=False, needs_layout_passes=False),
)


def kernel(poss_node, poss_edge, groundTruth, mask, edges):
    gt32 = groundTruth.astype(jnp.int32)
    maskf = mask.astype(jnp.float32)
    rect = jnp.concatenate([
        poss_edge[:_N],
        gt32.astype(jnp.float32)[:, None],
        maskf[:, None],
        jnp.zeros((_N, _REC - _ROW - 2), jnp.float32),
    ], axis=1)
    edges3 = edges.T.reshape(2, _E // _SUB, _SUB)
    pnode = jnp.pad(poss_node, ((0, _NPAD - _N), (0, _ROW - _C)),
                    constant_values=1.0)
    gtp = jnp.pad(gt32, (0, _NPAD - _N)).reshape(_NW * _NSTEPS, 16)
    maskp = jnp.pad(maskf, (0, _NPAD - _N)).reshape(_NW * _NSTEPS, 16)

    parts = _sc_call(poss_edge, rect, edges3, pnode, gtp, maskp)
    s2 = jnp.sum(parts[:, 0, :])
    slog = jnp.sum(parts[:, 1, :])
    den = jnp.sum(parts[:, 2, :])
    slast = jnp.sum(parts[:, 3, :])
    nlog = jnp.sum(parts[:, 4, :])
    ncnt = jnp.sum(parts[:, 5, :])

    loss = -nlog / ncnt
    semi = jnp.float32(_SEMI_LAMBDA) * (jnp.float32(_E) - slast) * s2
    el = -slog * jnp.float32(_EDGE_LAMBDA) / den
    el = el * jnp.float32(_EDGE_LAMBDA) / den
    return loss + semi + el


# speedup vs baseline: 55.4534x; 1.5039x over previous
import functools

import jax
import jax.numpy as jnp
from jax import lax
from jax.experimental import pallas as pl
from jax.experimental.pallas import tpu as pltpu
from jax.experimental.pallas import tpu_sc as plsc

_N = 100000
_E = 3200000
_C = 7
_ROW = 8
_REC = 16
_SEMI_LAMBDA = 0.001
_EDGE_LAMBDA = 1.0

_NC = 2
_NS = 16
_NW = _NC * _NS
_EPW = _E // _NW
_B = 800
_NCHUNK = _EPW // _B
_NPAIR = (_NCHUNK - 1) // 2
_SUB = 100
_NSUB = _B // _SUB
_STEPS = _B // 16
_NPT = 3136
_NPAD = _NW * _NPT
_NSTEPS = _NPT // 16

_LN2 = 0.6931471805599453


def _vlog(x):
    bits = lax.bitcast_convert_type(x, jnp.int32)
    e = (bits >> 23) - 127
    m = lax.bitcast_convert_type(
        (bits & jnp.int32(0x007FFFFF)) | jnp.int32(0x3F800000), jnp.float32)
    big = m > jnp.float32(1.41421356)
    m = jnp.where(big, m * jnp.float32(0.5), m)
    e = jnp.where(big, e + 1, e)
    f = m - jnp.float32(1.0)
    z = f * f
    p = jnp.float32(7.0376836292e-2)
    p = p * f + jnp.float32(-1.1514610310e-1)
    p = p * f + jnp.float32(1.1676998740e-1)
    p = p * f + jnp.float32(-1.2420140846e-1)
    p = p * f + jnp.float32(1.4249322787e-1)
    p = p * f + jnp.float32(-1.6668057665e-1)
    p = p * f + jnp.float32(2.0000714765e-1)
    p = p * f + jnp.float32(-2.4999993993e-1)
    p = p * f + jnp.float32(3.3333331174e-1)
    r = f * z * p - jnp.float32(0.5) * z + f
    return r + e.astype(jnp.float32) * jnp.float32(_LN2)


def _body(*refs):
    (pe, rect, edg, pnd, gtt, mskt, out) = refs[:7]
    (idx00, idx01, idx10, idx11, own0, own1,
     rec00, rec01, rec10, rec11,
     npn, ngt, nmsk, accout,
     semi0, semi1, semg0, semg1) = refs[7:]

    wid = lax.axis_index("s") * _NC + lax.axis_index("c")
    la = lax.iota(jnp.int32, 16)
    cols = [jnp.full((16,), c, jnp.int32) for c in range(10)]
    zero = jnp.zeros((16,), jnp.float32)
    one = jnp.full((16,), 1.0, jnp.float32)

    ebase = wid * _EPW
    rbase = wid * (_EPW // _SUB)

    def start_idx(ci, i0buf, i1buf, sem):
        erow = rbase + ci * _NSUB
        pltpu.async_copy(edg.at[0, pl.ds(erow, _NSUB)], i0buf, sem)
        pltpu.async_copy(edg.at[1, pl.ds(erow, _NSUB)], i1buf, sem)

    def start_own(ci, ownbuf, sem):
        pltpu.async_copy(pe.at[pl.ds(ebase + ci * _B, _B)], ownbuf, sem)

    def wait_fetch(i0buf, i1buf, ownbuf, sem):
        pltpu.make_async_copy(edg.at[0, pl.ds(0, _NSUB)], i0buf, sem).wait()
        pltpu.make_async_copy(edg.at[0, pl.ds(0, _NSUB)], i1buf, sem).wait()
        pltpu.make_async_copy(pe.at[pl.ds(0, _B)], ownbuf, sem).wait()

    def issue_gathers(i0buf, i1buf, r0buf, r1buf, sem):
        for j in range(_NSUB):
            pltpu.async_copy(
                rect.at[i0buf.at[j]], r0buf.at[pl.ds(j * _SUB, _SUB)], sem)
            pltpu.async_copy(
                rect.at[i1buf.at[j]], r1buf.at[pl.ds(j * _SUB, _SUB)], sem)

    def wait_gathers(r0buf, r1buf, sem):
        pltpu.make_async_copy(rect.at[pl.ds(0, _B)], r0buf, sem).wait()
        pltpu.make_async_copy(rect.at[pl.ds(0, _B)], r1buf, sem).wait()

    def compute(own, rec0, rec1, carry):
        def step(t, c2):
            s2, slog, sden, slast = c2
            j = t * 16 + la
            acc = jnp.zeros((16,), jnp.float32)
            for c in range(_ROW):
                a = plsc.load_gather(rec0, [j, cols[c]])
                b = plsc.load_gather(rec1, [j, cols[c]])
                d = a - b
                acc = acc + d * d
            g0f = plsc.load_gather(rec0, [j, cols[8]])
            m0f = plsc.load_gather(rec0, [j, cols[9]])
            g1f = plsc.load_gather(rec1, [j, cols[8]])
            m1f = plsc.load_gather(rec1, [j, cols[9]])
            g0 = g0f.astype(jnp.int32)
            g1 = g1f.astype(jnp.int32)
            plast = plsc.load_gather(own, [j, cols[7]])
            p0 = plsc.load_gather(own, [j, g0])
            p1 = plsc.load_gather(own, [j, g1])
            m0b = m0f > jnp.float32(0.5)
            m1b = m1f > jnp.float32(0.5)
            bothb = (m0f * m1f) > jnp.float32(0.5)
            anyf = jnp.maximum(m0f, m1f)
            sameb = g0 == g1
            arg1 = jnp.where(sameb, p0, plast)
            arg2 = jnp.where(m0b, plast + p0,
                             jnp.where(m1b, plast + p1, one))
            arg = jnp.where(bothb, arg1, arg2)
            return (s2 + acc, slog + _vlog(arg), sden + anyf, slast + plast)

        return lax.fori_loop(0, _STEPS, step, carry)

    start_idx(0, idx00, idx01, semi0)
    start_own(0, own0, semi0)

    nbase = wid * _NPT
    pltpu.sync_copy(pnd.at[pl.ds(nbase, _NPT)], npn)
    pltpu.sync_copy(gtt.at[pl.ds(wid * _NSTEPS, _NSTEPS)], ngt)
    pltpu.sync_copy(mskt.at[pl.ds(wid * _NSTEPS, _NSTEPS)], nmsk)

    def nstep(t, carry):
        nlog, ncnt = carry
        j = t * 16 + la
        g = ngt[t]
        pn = plsc.load_gather(npn, [j, g])
        mv = nmsk[t]
        return nlog + mv * _vlog(pn), ncnt + mv

    nlog, ncnt = lax.fori_loop(0, _NSTEPS, nstep, (zero, zero))

    wait_fetch(idx00, idx01, own0, semi0)
    issue_gathers(idx00, idx01, rec00, rec01, semg0)
    start_idx(1, idx10, idx11, semi1)
    start_own(1, own1, semi1)

    def pair(k, carry):
        c0 = 2 * k
        wait_fetch(idx10, idx11, own1, semi1)
        issue_gathers(idx10, idx11, rec10, rec11, semg1)
        wait_gathers(rec00, rec01, semg0)
        start_idx(c0 + 2, idx00, idx01, semi0)
        carry = compute(own0, rec00, rec01, carry)
        start_own(c0 + 2, own0, semi0)
        wait_fetch(idx00, idx01, own0, semi0)
        issue_gathers(idx00, idx01, rec00, rec01, semg0)
        wait_gathers(rec10, rec11, semg1)

        @pl.when(k < _NPAIR - 1)
        def _():
            start_idx(c0 + 3, idx10, idx11, semi1)
        carry = compute(own1, rec10, rec11, carry)

        @pl.when(k < _NPAIR - 1)
        def _():
            start_own(c0 + 3, own1, semi1)
        return carry

    carry = lax.fori_loop(0, _NPAIR, pair, (zero, zero, zero, zero))

    wait_gathers(rec00, rec01, semg0)
    s2, slog, sden, slast = compute(own0, rec00, rec01, carry)

    accout[0] = s2
    accout[1] = slog
    accout[2] = sden
    accout[3] = slast
    accout[4] = nlog
    accout[5] = ncnt
    pltpu.sync_copy(accout, out.at[wid])


_mesh = plsc.VectorSubcoreMesh(
    core_axis_name="c", subcore_axis_name="s", num_cores=_NC,
    num_subcores=_NS)

_sc_call = pl.kernel(
    _body,
    out_type=jax.ShapeDtypeStruct((_NW, 6, 16), jnp.float32),
    mesh=_mesh,
    scratch_types=(
        [pltpu.VMEM((_NSUB, _SUB), jnp.int32) for _ in range(4)]
        + [pltpu.VMEM((_B, _ROW), jnp.float32) for _ in range(2)]
        + [pltpu.VMEM((_B, _REC), jnp.float32) for _ in range(4)]
        + [
            pltpu.VMEM((_NPT, _ROW), jnp.float32),
            pltpu.VMEM((_NSTEPS, 16), jnp.int32),
            pltpu.VMEM((_NSTEPS, 16), jnp.float32),
            pltpu.VMEM((6, 16), jnp.float32),
        ]
        + [pltpu.SemaphoreType.DMA for _ in range(4)]),
    compiler_params=pltpu.CompilerParams(
        use_tc_tiling_on_sc=False, needs_layout_passes=False),
)


def kernel(poss_node, poss_edge, groundTruth, mask, edges):
    gt32 = groundTruth.astype(jnp.int32)
    maskf = mask.astype(jnp.float32)
    rect = jnp.concatenate([
        poss_edge[:_N],
        gt32.astype(jnp.float32)[:, None],
        maskf[:, None],
        jnp.zeros((_N, _REC - _ROW - 2), jnp.float32),
    ], axis=1)
    edges3 = edges.T.reshape(2, _E // _SUB, _SUB)
    pnode = jnp.pad(poss_node, ((0, _NPAD - _N), (0, _ROW - _C)),
                    constant_values=1.0)
    gtp = jnp.pad(gt32, (0, _NPAD - _N)).reshape(_NW * _NSTEPS, 16)
    maskp = jnp.pad(maskf, (0, _NPAD - _N)).reshape(_NW * _NSTEPS, 16)

    parts = _sc_call(poss_edge, rect, edges3, pnode, gtp, maskp)
    s2 = jnp.sum(parts[:, 0, :])
    slog = jnp.sum(parts[:, 1, :])
    den = jnp.sum(parts[:, 2, :])
    slast = jnp.sum(parts[:, 3, :])
    nlog = jnp.sum(parts[:, 4, :])
    ncnt = jnp.sum(parts[:, 5, :])

    loss = -nlog / ncnt
    semi = jnp.float32(_SEMI_LAMBDA) * (jnp.float32(_E) - slast) * s2
    el = -slog * jnp.float32(_EDGE_LAMBDA) / den
    el = el * jnp.float32(_EDGE_LAMBDA) / den
    return loss + semi + el
